# baseline (speedup 1.0000x reference)
import functools

import jax
import jax.numpy as jnp
from jax import lax
from jax.experimental import pallas as pl
from jax.experimental.pallas import tpu as pltpu
from jax.experimental.pallas import tpu_sc as plsc

B, S, H = 1, 2048, 1024
E = 8
NSLOT = 9
K = 2
FF = 2048
T = B * S

BLK = 256
PR = K * T + E * BLK
NBR = PR // BLK
SINK = PR
PPAD = PR + 16

NC, NS, L = 2, 16, 16
TPW = T // NS
NCHUNK = TPW // L

_F32 = jnp.float32
_I32 = jnp.int32


def _router_body(x_ref, rw_ref, out_ref, z_ref):
    x = x_ref[...]
    rw = rw_ref[...]
    logits = lax.dot_general(x, rw, (((1,), (1,)), ((), ())),
                             preferred_element_type=_F32)
    lane = lax.broadcasted_iota(_I32, (T, 16), 1)
    valid = lane < NSLOT
    masked = jnp.where(valid, logits, -1e30)
    m = jnp.max(masked, axis=1, keepdims=True)
    p = jnp.where(valid, jnp.exp(masked - m), 0.0)
    probs = p / jnp.sum(p, axis=1, keepdims=True)
    m1 = jnp.max(probs, axis=1, keepdims=True)
    c1 = jnp.min(jnp.where(probs == m1, lane, 99), axis=1, keepdims=True)
    probs2 = jnp.where(lane == c1, -1.0, probs)
    m2 = jnp.max(probs2, axis=1, keepdims=True)
    c2 = jnp.min(jnp.where(probs2 == m2, lane, 99), axis=1, keepdims=True)
    sw = m1 + m2 + 1e-6
    w1 = m1 / sw
    w2 = m2 / sw
    z_ref[0, 0] = jnp.sum(logits * logits)
    lane128 = lax.broadcasted_iota(_I32, (T, 128), 1)
    def bc(v):
        return jnp.broadcast_to(v, (T, 128))
    out = jnp.where(lane128 == 0, bc(c1.astype(_F32)),
          jnp.where(lane128 == 1, bc(c2.astype(_F32)),
          jnp.where(lane128 == 2, bc(w1),
          jnp.where(lane128 == 3, bc(w2), 0.0))))
    out_ref[...] = out


def _run_router(x2d, router_w):
    rw16 = jnp.zeros((16, H), _F32).at[:NSLOT].set(router_w)
    return pl.pallas_call(
        _router_body,
        out_shape=[jax.ShapeDtypeStruct((T, 128), _F32),
                   jax.ShapeDtypeStruct((1, 1), _F32)],
        out_specs=[pl.BlockSpec(memory_space=pltpu.VMEM),
                   pl.BlockSpec(memory_space=pltpu.SMEM)],
    )(x2d, rw16)


def _splat(vec, e):
    io = lax.iota(_I32, L)
    return jnp.sum(jnp.where(io == e, vec, jnp.zeros_like(vec)))


def _routing_body(e1_hbm, e2_hbm, w1_hbm, w2_hbm,
                  gidx_hbm, wrow_hbm, pos1_hbm, pos2_hbm,
                  bexp_hbm, bact_hbm, counts_hbm,
                  e1v, e2v, w1v, w2v, hrow, gridv,
                  posb1, posb2, idxb1, idxb2, tokb,
                  padidx, padzf, padzi, outv, sem, grid_sp):
    c = lax.axis_index("c")
    s = lax.axis_index("s")
    io = lax.iota(_I32, L)

    @pl.when(c == 0)
    def _work():
        base_tok = s * TPW
        pltpu.sync_copy(e1_hbm.at[pl.ds(base_tok, TPW)], e1v)
        pltpu.sync_copy(e2_hbm.at[pl.ds(base_tok, TPW)], e2v)
        pltpu.sync_copy(w1_hbm.at[pl.ds(base_tok, TPW)], w1v)
        pltpu.sync_copy(w2_hbm.at[pl.ds(base_tok, TPW)], w2v)

        hv = jnp.zeros((L,), _I32)
        for src in (e1v, e2v):
            for i in range(NCHUNK):
                a = src[pl.ds(i * L, L)]
                for e in range(E):
                    pc = plsc.all_reduce_population_count(a == e)
                    hv = hv + jnp.where(io == e, pc, 0)
        hrow[...] = hv
        pltpu.sync_copy(hrow, grid_sp.at[pl.ds(s * L, L)])
        plsc.subcore_barrier()
        pltpu.sync_copy(grid_sp, gridv)

        counts = jnp.zeros((L,), _I32)
        mybase = jnp.zeros((L,), _I32)
        for t in range(NS):
            row = gridv[pl.ds(t * L, L)]
            counts = counts + row
            mybase = mybase + jnp.where(jnp.full((L,), t) < s, row, 0)

        cnt1 = counts + jnp.where(io == 0, 1, 0)
        padded = jnp.where(io < E, ((cnt1 + (BLK - 1)) // BLK) * BLK, 0)
        start = plsc.cumsum(padded) - padded
        dummy = _splat(start + counts, 0)

        cnt_run = jnp.zeros((L,), _I32)
        for slot, (srcE, posb, idxb) in enumerate(
                ((e1v, posb1, idxb1), (e2v, posb2, idxb2))):
            for i in range(NCHUNK):
                a = srcE[pl.ds(i * L, L)]
                posv = jnp.full((L,), 0) + dummy
                for e in range(E):
                    m = a == e
                    mi = m.astype(_I32)
                    excl = plsc.cumsum(mi) - mi
                    base_e = _splat(start + mybase + cnt_run, e)
                    posv = jnp.where(m, base_e + excl, posv)
                    cnt_run = cnt_run + jnp.where(
                        io == e, plsc.all_reduce_population_count(m), 0)
                posb[pl.ds(i * L, L)] = posv
                idxb[pl.ds(i * L, L)] = jnp.where(a < E, posv,
                                                  jnp.full((L,), SINK))
                if slot == 0:
                    tokb[pl.ds(i * L, L)] = base_tok + i * L + io

        pltpu.sync_copy(posb1, pos1_hbm.at[pl.ds(base_tok, TPW)])
        pltpu.sync_copy(posb2, pos2_hbm.at[pl.ds(base_tok, TPW)])
        pltpu.async_copy(tokb, gidx_hbm.at[idxb1], sem).wait()
        pltpu.async_copy(tokb, gidx_hbm.at[idxb2], sem).wait()
        pltpu.async_copy(w1v, wrow_hbm.at[idxb1], sem).wait()
        pltpu.async_copy(w2v, wrow_hbm.at[idxb2], sem).wait()

        @pl.when(s < E)
        def _pads():
            lo = _splat(start + counts, s)
            hi = _splat(start + padded, s)
            padzf[...] = jnp.zeros((L,), _F32)
            padzi[...] = jnp.zeros((L,), _I32)
            for j in range((2 * BLK) // L):
                idxv = lo + j * L + io
                padidx[...] = jnp.where(idxv < hi, idxv, jnp.full((L,), SINK))
                pltpu.async_copy(padzf, wrow_hbm.at[padidx], sem).wait()
                pltpu.async_copy(padzi, gidx_hbm.at[padidx], sem).wait()

        @pl.when(s == E)
        def _bmeta():
            for half in range(2):
                b = half * L + io
                rowstart = b * BLK
                expv = jnp.zeros((L,), _I32)
                actv = jnp.zeros((L,), _I32)
                for e in range(E):
                    st_e = _splat(start, e)
                    pd_e = _splat(padded, e)
                    m = (rowstart >= st_e) & (rowstart < st_e + pd_e)
                    expv = jnp.where(m, e, expv)
                    actv = jnp.where(m, 1, actv)
                outv[...] = expv
                pltpu.sync_copy(outv, bexp_hbm.at[pl.ds(half * L, L)])
                outv[...] = actv
                pltpu.sync_copy(outv, bact_hbm.at[pl.ds(half * L, L)])

        @pl.when(s == E + 1)
        def _counts():
            outv[...] = counts
            pltpu.sync_copy(outv, counts_hbm)


def _run_routing(e1, e2, w1, w2):
    mesh = plsc.VectorSubcoreMesh(core_axis_name="c", subcore_axis_name="s",
                                  num_cores=NC, num_subcores=NS)
    f = pl.kernel(
        _routing_body,
        out_type=[
            jax.ShapeDtypeStruct((PPAD,), _I32),
            jax.ShapeDtypeStruct((PPAD,), _F32),
            jax.ShapeDtypeStruct((T,), _I32),
            jax.ShapeDtypeStruct((T,), _I32),
            jax.ShapeDtypeStruct((2 * L,), _I32),
            jax.ShapeDtypeStruct((2 * L,), _I32),
            jax.ShapeDtypeStruct((L,), _I32),
        ],
        mesh=mesh,
        compiler_params=pltpu.CompilerParams(needs_layout_passes=False),
        scratch_types=[
            pltpu.VMEM((TPW,), _I32), pltpu.VMEM((TPW,), _I32),
            pltpu.VMEM((TPW,), _F32), pltpu.VMEM((TPW,), _F32),
            pltpu.VMEM((L,), _I32), pltpu.VMEM((NS * L,), _I32),
            pltpu.VMEM((TPW,), _I32), pltpu.VMEM((TPW,), _I32),
            pltpu.VMEM((TPW,), _I32), pltpu.VMEM((TPW,), _I32),
            pltpu.VMEM((TPW,), _I32),
            pltpu.VMEM((L,), _I32), pltpu.VMEM((L,), _F32),
            pltpu.VMEM((L,), _I32), pltpu.VMEM((L,), _I32),
            pltpu.SemaphoreType.DMA,
            pltpu.VMEM_SHARED((NS * L,), _I32),
        ],
    )
    return f(e1, e2, w1, w2)


def _gather_body(x_hbm, gidx_hbm, xd_hbm, idxv, rows, sem):
    wid = lax.axis_index("s") * NC + lax.axis_index("c")
    rpw = PR // (NC * NS)
    csz = 64
    for ch in range(rpw // csz):
        r0 = wid * rpw + ch * csz
        pltpu.sync_copy(gidx_hbm.at[pl.ds(r0, csz)], idxv)
        for j in range(csz // L):
            v = idxv[pl.ds(j * L, L)]
            idxv[pl.ds(j * L, L)] = jnp.clip(v, 0, T - 1)
        pltpu.async_copy(x_hbm.at[idxv], rows, sem).wait()
        pltpu.sync_copy(rows, xd_hbm.at[pl.ds(r0, csz)])


def _run_gather(x2d, gidx):
    mesh = plsc.VectorSubcoreMesh(core_axis_name="c", subcore_axis_name="s",
                                  num_cores=NC, num_subcores=NS)
    f = pl.kernel(
        _gather_body,
        out_type=[jax.ShapeDtypeStruct((PR, H), _F32)],
        mesh=mesh,
        compiler_params=pltpu.CompilerParams(needs_layout_passes=False),
        scratch_types=[
            pltpu.VMEM((64,), _I32),
            pltpu.VMEM((64, H), _F32),
            pltpu.SemaphoreType.DMA,
        ],
    )
    return f(x2d, gidx)[0]


def _gffn_body(bexp_ref, bact_ref, xd_ref, gw_ref, uw_ref, dw_ref, wr_ref,
               yd_ref):
    i = pl.program_id(0)

    @pl.when(bact_ref[i] == 1)
    def _():
        xb = xd_ref[...]
        g = lax.dot_general(xb, gw_ref[0], (((1,), (1,)), ((), ())),
                            preferred_element_type=_F32)
        u = lax.dot_general(xb, uw_ref[0], (((1,), (1,)), ((), ())),
                            preferred_element_type=_F32)
        h = g * lax.logistic(g) * u
        y = lax.dot_general(h, dw_ref[0], (((1,), (1,)), ((), ())),
                            preferred_element_type=_F32)
        w = jnp.transpose(wr_ref[0])
        yd_ref[...] = y * w


def _run_gffn(bexp, bact, xd, gate_w, up_w, down_w, wrow):
    wr2d = wrow[:PR].reshape(NBR, 1, BLK)
    grid_spec = pltpu.PrefetchScalarGridSpec(
        num_scalar_prefetch=2,
        grid=(NBR,),
        in_specs=[
            pl.BlockSpec((BLK, H), lambda i, be, ba: (i, 0)),
            pl.BlockSpec((1, FF, H), lambda i, be, ba: (be[i], 0, 0)),
            pl.BlockSpec((1, FF, H), lambda i, be, ba: (be[i], 0, 0)),
            pl.BlockSpec((1, H, FF), lambda i, be, ba: (be[i], 0, 0)),
            pl.BlockSpec((1, 1, BLK), lambda i, be, ba: (i, 0, 0)),
        ],
        out_specs=pl.BlockSpec((BLK, H), lambda i, be, ba: (i, 0)),
    )
    return pl.pallas_call(
        _gffn_body,
        grid_spec=grid_spec,
        out_shape=jax.ShapeDtypeStruct((PR, H), _F32),
    )(bexp[:NBR], bact[:NBR], xd, gate_w, up_w, down_w, wr2d)


def _sffn_body(x_ref, gw_ref, uw_ref, dw_ref, o_ref):
    xb = x_ref[...]
    g = lax.dot_general(xb, gw_ref[...], (((1,), (1,)), ((), ())),
                        preferred_element_type=_F32)
    u = lax.dot_general(xb, uw_ref[...], (((1,), (1,)), ((), ())),
                        preferred_element_type=_F32)
    h = g * lax.logistic(g) * u
    o_ref[...] = lax.dot_general(h, dw_ref[...], (((1,), (1,)), ((), ())),
                                 preferred_element_type=_F32)


def _run_sffn(x2d, sgw, suw, sdw):
    return pl.pallas_call(
        _sffn_body,
        grid=(T // BLK,),
        in_specs=[
            pl.BlockSpec((BLK, H), lambda i: (i, 0)),
            pl.BlockSpec((FF, H), lambda i: (0, 0)),
            pl.BlockSpec((FF, H), lambda i: (0, 0)),
            pl.BlockSpec((H, FF), lambda i: (0, 0)),
        ],
        out_specs=pl.BlockSpec((BLK, H), lambda i: (i, 0)),
        out_shape=jax.ShapeDtypeStruct((T, H), _F32),
    )(x2d, sgw, suw, sdw)


def _combine_body(yds_hbm, yd_hbm, pos1_hbm, pos2_hbm, out_hbm,
                  p1v, p2v, r1, r2, acc, sem):
    wid = lax.axis_index("s") * NC + lax.axis_index("c")
    tpw = T // (NC * NS)
    csz = 16
    for ch in range(tpw // csz):
        t0 = wid * tpw + ch * csz
        pltpu.sync_copy(pos1_hbm.at[pl.ds(t0, csz)], p1v)
        pltpu.sync_copy(pos2_hbm.at[pl.ds(t0, csz)], p2v)
        cp1 = pltpu.async_copy(yd_hbm.at[p1v], r1, sem)
        cp2 = pltpu.async_copy(yd_hbm.at[p2v], r2, sem)
        pltpu.sync_copy(yds_hbm.at[pl.ds(t0, csz)], acc)
        cp1.wait()
        cp2.wait()

        def add_row(r, _):
            for k in range(H // L):
                a = (acc[r, pl.ds(k * L, L)] + r1[r, pl.ds(k * L, L)]
                     + r2[r, pl.ds(k * L, L)])
                acc[r, pl.ds(k * L, L)] = a
            return 0

        lax.fori_loop(0, csz, add_row, 0)
        pltpu.sync_copy(acc, out_hbm.at[pl.ds(t0, csz)])


def _run_combine(yds, yd, pos1, pos2):
    mesh = plsc.VectorSubcoreMesh(core_axis_name="c", subcore_axis_name="s",
                                  num_cores=NC, num_subcores=NS)
    f = pl.kernel(
        _combine_body,
        out_type=[jax.ShapeDtypeStruct((T, H), _F32)],
        mesh=mesh,
        compiler_params=pltpu.CompilerParams(needs_layout_passes=False),
        scratch_types=[
            pltpu.VMEM((16,), _I32), pltpu.VMEM((16,), _I32),
            pltpu.VMEM((16, H), _F32), pltpu.VMEM((16, H), _F32),
            pltpu.VMEM((16, H), _F32),
            pltpu.SemaphoreType.DMA,
        ],
    )
    return f(yds, yd, pos1, pos2)[0]


def kernel(hidden_states, router_w, gate_w, up_w, down_w,
           shared_gate_w, shared_up_w, shared_down_w):
    x2d = hidden_states.reshape(T, H)

    rtr, z_sum = _run_router(x2d, router_w)
    e1 = rtr[:, 0].astype(_I32)
    e2 = rtr[:, 1].astype(_I32)
    w1 = rtr[:, 2]
    w2 = rtr[:, 3]

    gidx, wrow, pos1, pos2, bexp, bact, counts = _run_routing(e1, e2, w1, w2)
    xd = _run_gather(x2d, gidx)
    yd = _run_gffn(bexp, bact, xd, gate_w, up_w, down_w, wrow)
    yds = _run_sffn(x2d, shared_gate_w, shared_up_w, shared_down_w)
    out2d = _run_combine(yds, yd, pos1, pos2)

    final_outputs = out2d.reshape(B, S, H)

    loads = jnp.concatenate(
        [counts[:E].astype(_F32), jnp.full((1,), float(T), _F32)])
    loads_norm = loads / (jnp.sum(loads) + 1e-6)
    ideal = 1.0 / NSLOT
    load_balance_loss = jnp.mean((loads_norm - ideal) ** 2)
    router_z_loss = z_sum[0, 0] / T
    total_aux_loss = 0.01 * load_balance_loss + 0.01 * router_z_loss
    return final_outputs, total_aux_loss


# speedup vs baseline: 3.3080x; 3.3080x over previous
import jax
import jax.numpy as jnp
from jax import lax
from jax.experimental import pallas as pl
from jax.experimental.pallas import tpu as pltpu
from jax.experimental.pallas import tpu_sc as plsc

B, S, H = 1, 2048, 1024
E = 8
NSLOT = 9
FF = 2048
T = B * S

BLK = 256
PR = 2 * T + E * BLK
NBR = PR // BLK
SINK = PR
PPAD = PR + 16

NC, NS, L = 2, 16, 16
NW = NC * NS
TPW = T // NW

_F32 = jnp.float32
_I32 = jnp.int32


def _router_body(x_ref, rw_ref, out_ref, meta_ref, z_ref):
    x = x_ref[...]
    rw = rw_ref[...]
    logits = lax.dot_general(x, rw, (((1,), (1,)), ((), ())),
                             preferred_element_type=_F32)
    lane = lax.broadcasted_iota(_I32, (T, 16), 1)
    valid = lane < NSLOT
    masked = jnp.where(valid, logits, -1e30)
    m = jnp.max(masked, axis=1, keepdims=True)
    p = jnp.where(valid, jnp.exp(masked - m), 0.0)
    probs = p / jnp.sum(p, axis=1, keepdims=True)
    m1 = jnp.max(probs, axis=1, keepdims=True)
    c1 = jnp.min(jnp.where(probs == m1, lane, 99), axis=1, keepdims=True)
    probs2 = jnp.where(lane == c1, -1.0, probs)
    m2 = jnp.max(probs2, axis=1, keepdims=True)
    c2 = jnp.min(jnp.where(probs2 == m2, lane, 99), axis=1, keepdims=True)
    sw = m1 + m2 + 1e-6
    w1 = m1 / sw
    w2 = m2 / sw
    z_ref[0, 0] = jnp.sum(logits * logits)

    mm1 = (lane == c1).astype(_F32)
    mm2 = (lane == c2).astype(_F32)
    mm = mm1 + mm2
    r = lax.broadcasted_iota(_I32, (T, T), 0)
    cc = lax.broadcasted_iota(_I32, (T, T), 1)
    lt = (cc < r).astype(_F32)
    cex = lax.dot_general(lt, mm, (((1,), (0,)), ((), ())),
                          preferred_element_type=_F32)
    counts = cex[T - 1:T, :] + mm[T - 1:T, :]

    lane_r = lane[0:1, :]
    cnt1 = counts + (lane_r == 0).astype(_F32)
    padded = jnp.where(lane_r < E,
                       jnp.ceil(cnt1 / BLK) * BLK, 0.0)
    ltl = (lane[0:16, :] < lax.broadcasted_iota(_I32, (16, 16), 0))
    start = lax.dot_general(padded, ltl.astype(_F32),
                            (((1,), (0,)), ((), ())),
                            preferred_element_type=_F32)

    pos_base = start + cex
    pos1 = jnp.sum(pos_base * mm1, axis=1, keepdims=True)
    pos2 = jnp.sum(pos_base * mm2, axis=1, keepdims=True)
    dummy = jnp.sum((start + counts) * (lane_r == 0).astype(_F32))
    sinkf = float(SINK)
    is_r1 = c1 < E
    is_r2 = c2 < E
    cpos1 = jnp.where(is_r1, pos1, dummy)
    cpos2 = jnp.where(is_r2, pos2, dummy)
    idx1 = jnp.where(is_r1, pos1, sinkf)
    idx2 = jnp.where(is_r2, pos2, sinkf)

    lane128 = lax.broadcasted_iota(_I32, (T, 128), 1)
    def bc(v):
        return jnp.broadcast_to(v, (T, 128))
    out = jnp.where(lane128 == 0, bc(c1.astype(_F32)),
          jnp.where(lane128 == 1, bc(c2.astype(_F32)),
          jnp.where(lane128 == 2, bc(w1),
          jnp.where(lane128 == 3, bc(w2),
          jnp.where(lane128 == 4, bc(cpos1),
          jnp.where(lane128 == 5, bc(cpos2),
          jnp.where(lane128 == 6, bc(idx1),
          jnp.where(lane128 == 7, bc(idx2), 0.0))))))))
    out_ref[...] = out

    bl = lax.broadcasted_iota(_I32, (16, 128), 1).astype(_F32) * BLK
    startc = jnp.broadcast_to(jnp.transpose(start), (16, 128))
    paddedc = jnp.broadcast_to(jnp.transpose(padded), (16, 128))
    lane16c = lax.broadcasted_iota(_I32, (16, 128), 0)
    inseg = ((bl >= startc) & (bl < startc + paddedc)
             & (lane16c < E)).astype(_F32)
    eidx = lane16c.astype(_F32) * inseg
    ones16 = jnp.ones((1, 16), _F32)
    bexp = lax.dot_general(ones16, eidx, (((1,), (0,)), ((), ())),
                           preferred_element_type=_F32)
    bact = lax.dot_general(ones16, inseg, (((1,), (0,)), ((), ())),
                           preferred_element_type=_F32)
    eye = (lax.broadcasted_iota(_I32, (16, 128), 0)
           == lax.broadcasted_iota(_I32, (16, 128), 1)).astype(_F32)
    counts128 = lax.dot_general(counts, eye, (((1,), (0,)), ((), ())),
                                preferred_element_type=_F32)
    row8 = lax.broadcasted_iota(_I32, (8, 128), 0)
    meta = jnp.where(row8 == 0, jnp.broadcast_to(bexp, (8, 128)),
           jnp.where(row8 == 1, jnp.broadcast_to(bact, (8, 128)),
           jnp.where(row8 == 2, jnp.broadcast_to(counts128, (8, 128)),
           jnp.where(row8 == 3,
                     jnp.where(lane128[0:8, :] == 0, dummy, sinkf), 0.0))))
    meta_ref[...] = meta


def _run_router(x2d, router_w):
    rw16 = jnp.zeros((16, H), _F32).at[:NSLOT].set(router_w)
    return pl.pallas_call(
        _router_body,
        out_shape=[jax.ShapeDtypeStruct((T, 128), _F32),
                   jax.ShapeDtypeStruct((8, 128), _F32),
                   jax.ShapeDtypeStruct((1, 1), _F32)],
        out_specs=[pl.BlockSpec(memory_space=pltpu.VMEM),
                   pl.BlockSpec(memory_space=pltpu.VMEM),
                   pl.BlockSpec(memory_space=pltpu.SMEM)],
    )(x2d, rw16)


def _dispatch_body(x_hbm, idx1_hbm, idx2_hbm, w1_hbm, w2_hbm, dmy_hbm,
                   z_hbm, xd_hbm, wrow_hbm,

                   i1r, i2r, w1v, w2v, rv, dmyv, zrow, zw, sem):
    wid = lax.axis_index("s") * NC + lax.axis_index("c")
    t0 = wid * TPW
    nch = TPW // L
    pltpu.sync_copy(idx1_hbm.at[pl.ds(wid * nch, nch)], i1r)
    pltpu.sync_copy(idx2_hbm.at[pl.ds(wid * nch, nch)], i2r)
    pltpu.sync_copy(w1_hbm.at[pl.ds(t0, TPW)], w1v)
    pltpu.sync_copy(w2_hbm.at[pl.ds(t0, TPW)], w2v)
    i1m = i1r
    i2m = i2r
    for j in range(TPW // L):
        pltpu.sync_copy(x_hbm.at[pl.ds(t0 + j * L, L)], rv)
        cpa = pltpu.async_copy(rv, xd_hbm.at[i1m.at[j]], sem)
        cpb = pltpu.async_copy(rv, xd_hbm.at[i2m.at[j]], sem)
        cpc = pltpu.async_copy(w1v.at[pl.ds(j * L, L)],
                               wrow_hbm.at[i1m.at[j]], sem)
        cpd = pltpu.async_copy(w2v.at[pl.ds(j * L, L)],
                               wrow_hbm.at[i2m.at[j]], sem)
        cpa.wait()
        cpb.wait()
        cpc.wait()
        cpd.wait()

    @pl.when(wid == 0)
    def _dummy():
        pltpu.sync_copy(dmy_hbm, dmyv)
        pltpu.sync_copy(z_hbm, zrow)
        pltpu.sync_copy(z_hbm.at[0, pl.ds(0, L)], zw)
        pltpu.async_copy(zrow, xd_hbm.at[dmyv], sem).wait()
        pltpu.async_copy(zw, wrow_hbm.at[dmyv], sem).wait()


def _run_dispatch(x2d, idx1, idx2, w1, w2, dmy):
    mesh = plsc.VectorSubcoreMesh(core_axis_name="c", subcore_axis_name="s",
                                  num_cores=NC, num_subcores=NS)
    f = pl.kernel(
        _dispatch_body,
        out_type=[jax.ShapeDtypeStruct((PPAD, H), _F32),
                  jax.ShapeDtypeStruct((PPAD,), _F32)],
        mesh=mesh,
        compiler_params=pltpu.CompilerParams(needs_layout_passes=False),
        scratch_types=[
            pltpu.VMEM((TPW // L, L), _I32), pltpu.VMEM((TPW // L, L), _I32),
            pltpu.VMEM((TPW,), _F32), pltpu.VMEM((TPW,), _F32),
            pltpu.VMEM((L, H), _F32),
            pltpu.VMEM((L,), _I32),
            pltpu.VMEM((L, H), _F32),
            pltpu.VMEM((L,), _F32),
            pltpu.SemaphoreType.DMA,
        ],
    )
    return f(x2d, idx1.reshape(T // L, L), idx2.reshape(T // L, L),
             w1, w2, dmy, jnp.zeros((L, H), _F32))


def _gffn_body(bexp_ref, bact_ref, xd_ref, gw_ref, uw_ref, dw_ref, wr_ref,
               yd_ref):
    i = pl.program_id(0)

    @pl.when(bact_ref[i] == 1)
    def _():
        xb = xd_ref[...]
        g = lax.dot_general(xb, gw_ref[0], (((1,), (1,)), ((), ())),
                            preferred_element_type=_F32)
        u = lax.dot_general(xb, uw_ref[0], (((1,), (1,)), ((), ())),
                            preferred_element_type=_F32)
        h = g * lax.logistic(g) * u
        y = lax.dot_general(h, dw_ref[0], (((1,), (1,)), ((), ())),
                            preferred_element_type=_F32)
        w = jnp.transpose(wr_ref[0])
        yd_ref[...] = y * w


def _run_gffn(bexp, bact, xd, gate_w, up_w, down_w, wrow):
    wr3d = wrow[:PR].reshape(NBR, 1, BLK)
    grid_spec = pltpu.PrefetchScalarGridSpec(
        num_scalar_prefetch=2,
        grid=(NBR,),
        in_specs=[
            pl.BlockSpec((BLK, H), lambda i, be, ba: (i, 0)),
            pl.BlockSpec((1, FF, H), lambda i, be, ba: (be[i], 0, 0)),
            pl.BlockSpec((1, FF, H), lambda i, be, ba: (be[i], 0, 0)),
            pl.BlockSpec((1, H, FF), lambda i, be, ba: (be[i], 0, 0)),
            pl.BlockSpec((1, 1, BLK), lambda i, be, ba: (i, 0, 0)),
        ],
        out_specs=pl.BlockSpec((BLK, H), lambda i, be, ba: (i, 0)),
    )
    return pl.pallas_call(
        _gffn_body,
        grid_spec=grid_spec,
        out_shape=jax.ShapeDtypeStruct((PR, H), _F32),
    )(bexp, bact, xd, gate_w, up_w, down_w, wr3d)


def _sffn_body(x_ref, gw_ref, uw_ref, dw_ref, o_ref):
    xb = x_ref[...]
    g = lax.dot_general(xb, gw_ref[...], (((1,), (1,)), ((), ())),
                        preferred_element_type=_F32)
    u = lax.dot_general(xb, uw_ref[...], (((1,), (1,)), ((), ())),
                        preferred_element_type=_F32)
    h = g * lax.logistic(g) * u
    o_ref[...] = lax.dot_general(h, dw_ref[...], (((1,), (1,)), ((), ())),
                                 preferred_element_type=_F32)


def _run_sffn(x2d, sgw, suw, sdw):
    return pl.pallas_call(
        _sffn_body,
        grid=(T // BLK,),
        in_specs=[
            pl.BlockSpec((BLK, H), lambda i: (i, 0)),
            pl.BlockSpec((FF, H), lambda i: (0, 0)),
            pl.BlockSpec((FF, H), lambda i: (0, 0)),
            pl.BlockSpec((H, FF), lambda i: (0, 0)),
        ],
        out_specs=pl.BlockSpec((BLK, H), lambda i: (i, 0)),
        out_shape=jax.ShapeDtypeStruct((T, H), _F32),
    )(x2d, sgw, suw, sdw)


def _combine_body(yds_hbm, yd_hbm, pos1_hbm, pos2_hbm, out_hbm,
                  p1v, p2v, r1, r2, acc, sem):
    wid = lax.axis_index("s") * NC + lax.axis_index("c")
    csz = 16
    for ch in range(TPW // csz):
        t0 = wid * TPW + ch * csz
        pltpu.sync_copy(pos1_hbm.at[pl.ds(t0, csz)], p1v)
        pltpu.sync_copy(pos2_hbm.at[pl.ds(t0, csz)], p2v)
        cp1 = pltpu.async_copy(yd_hbm.at[p1v], r1, sem)
        cp2 = pltpu.async_copy(yd_hbm.at[p2v], r2, sem)
        pltpu.sync_copy(yds_hbm.at[pl.ds(t0, csz)], acc)
        cp1.wait()
        cp2.wait()

        def add_row(r, _):
            for k in range(H // L):
                a = (acc[r, pl.ds(k * L, L)] + r1[r, pl.ds(k * L, L)]
                     + r2[r, pl.ds(k * L, L)])
                acc[r, pl.ds(k * L, L)] = a
            return 0

        lax.fori_loop(0, csz, add_row, 0)
        pltpu.sync_copy(acc, out_hbm.at[pl.ds(t0, csz)])


def _run_combine(yds, yd, pos1, pos2):
    mesh = plsc.VectorSubcoreMesh(core_axis_name="c", subcore_axis_name="s",
                                  num_cores=NC, num_subcores=NS)
    f = pl.kernel(
        _combine_body,
        out_type=[jax.ShapeDtypeStruct((T, H), _F32)],
        mesh=mesh,
        compiler_params=pltpu.CompilerParams(needs_layout_passes=False),
        scratch_types=[
            pltpu.VMEM((16,), _I32), pltpu.VMEM((16,), _I32),
            pltpu.VMEM((16, H), _F32), pltpu.VMEM((16, H), _F32),
            pltpu.VMEM((16, H), _F32),
            pltpu.SemaphoreType.DMA,
        ],
    )
    return f(yds, yd, pos1, pos2)[0]


def kernel(hidden_states, router_w, gate_w, up_w, down_w,
           shared_gate_w, shared_up_w, shared_down_w):
    x2d = hidden_states.reshape(T, H)

    rtr, meta, z_sum = _run_router(x2d, router_w)
    w1 = rtr[:, 2]
    w2 = rtr[:, 3]
    cpos1 = rtr[:, 4].astype(_I32)
    cpos2 = rtr[:, 5].astype(_I32)
    idx1 = rtr[:, 6].astype(_I32)
    idx2 = rtr[:, 7].astype(_I32)
    bexp = meta[0, :NBR].astype(_I32)
    bact = meta[1, :NBR].astype(_I32)
    counts = meta[2, :16]
    dmy = meta[3, :16].astype(_I32)

    xd, wrow = _run_dispatch(x2d, idx1, idx2, w1, w2, dmy)
    yd = _run_gffn(bexp, bact, xd, gate_w, up_w, down_w, wrow)
    yds = _run_sffn(x2d, shared_gate_w, shared_up_w, shared_down_w)
    out2d = _run_combine(yds, yd, cpos1, cpos2)

    final_outputs = out2d.reshape(B, S, H)

    loads = jnp.concatenate([counts[:E], jnp.full((1,), float(T), _F32)])
    loads_norm = loads / (jnp.sum(loads) + 1e-6)
    ideal = 1.0 / NSLOT
    load_balance_loss = jnp.mean((loads_norm - ideal) ** 2)
    router_z_loss = z_sum[0, 0] / T
    total_aux_loss = 0.01 * load_balance_loss + 0.01 * router_z_loss
    return final_outputs, total_aux_loss


# speedup vs baseline: 3.3342x; 1.0079x over previous
import jax
import jax.numpy as jnp
from jax import lax
from jax.experimental import pallas as pl
from jax.experimental.pallas import tpu as pltpu
from jax.experimental.pallas import tpu_sc as plsc

B, S, H = 1, 2048, 1024
E = 8
NSLOT = 9
FF = 2048
T = B * S

BLK = 256
PR = 2 * T + E * BLK
NBR = PR // BLK
SINK = PR
PPAD = PR + 16

NC, NS, L = 2, 16, 16
NW = NC * NS
TPW = T // NW

_F32 = jnp.float32
_I32 = jnp.int32


def _router_body(x_ref, rw_ref, out_ref, meta_ref, z_ref):
    x = x_ref[...]
    rw = rw_ref[...]
    logits = lax.dot_general(x, rw, (((1,), (1,)), ((), ())),
                             preferred_element_type=_F32)
    lane = lax.broadcasted_iota(_I32, (T, 16), 1)
    valid = lane < NSLOT
    masked = jnp.where(valid, logits, -1e30)
    m = jnp.max(masked, axis=1, keepdims=True)
    p = jnp.where(valid, jnp.exp(masked - m), 0.0)
    probs = p / jnp.sum(p, axis=1, keepdims=True)
    m1 = jnp.max(probs, axis=1, keepdims=True)
    c1 = jnp.min(jnp.where(probs == m1, lane, 99), axis=1, keepdims=True)
    probs2 = jnp.where(lane == c1, -1.0, probs)
    m2 = jnp.max(probs2, axis=1, keepdims=True)
    c2 = jnp.min(jnp.where(probs2 == m2, lane, 99), axis=1, keepdims=True)
    sw = m1 + m2 + 1e-6
    w1 = m1 / sw
    w2 = m2 / sw
    z_ref[0, 0] = jnp.sum(logits * logits)

    mm1 = (lane == c1).astype(_F32)
    mm2 = (lane == c2).astype(_F32)
    mm = mm1 + mm2
    r = lax.broadcasted_iota(_I32, (T, T), 0)
    cc = lax.broadcasted_iota(_I32, (T, T), 1)
    lt = (cc < r).astype(_F32)
    cex = lax.dot_general(lt, mm, (((1,), (0,)), ((), ())),
                          preferred_element_type=_F32)
    counts = cex[T - 1:T, :] + mm[T - 1:T, :]

    lane_r = lane[0:1, :]
    cnt1 = counts + (lane_r == 0).astype(_F32)
    padded = jnp.where(lane_r < E,
                       jnp.ceil(cnt1 / BLK) * BLK, 0.0)
    ltl = (lane[0:16, :] < lax.broadcasted_iota(_I32, (16, 16), 0))
    start = lax.dot_general(padded, ltl.astype(_F32),
                            (((1,), (0,)), ((), ())),
                            preferred_element_type=_F32)

    pos_base = start + cex
    pos1 = jnp.sum(pos_base * mm1, axis=1, keepdims=True)
    pos2 = jnp.sum(pos_base * mm2, axis=1, keepdims=True)
    dummy = jnp.sum((start + counts) * (lane_r == 0).astype(_F32))
    sinkf = float(SINK)
    is_r1 = c1 < E
    is_r2 = c2 < E
    cpos1 = jnp.where(is_r1, pos1, dummy)
    cpos2 = jnp.where(is_r2, pos2, dummy)
    idx1 = jnp.where(is_r1, pos1, sinkf)
    idx2 = jnp.where(is_r2, pos2, sinkf)

    lane128 = lax.broadcasted_iota(_I32, (T, 128), 1)
    def bc(v):
        return jnp.broadcast_to(v, (T, 128))
    out = jnp.where(lane128 == 0, bc(c1.astype(_F32)),
          jnp.where(lane128 == 1, bc(c2.astype(_F32)),
          jnp.where(lane128 == 2, bc(w1),
          jnp.where(lane128 == 3, bc(w2),
          jnp.where(lane128 == 4, bc(cpos1),
          jnp.where(lane128 == 5, bc(cpos2),
          jnp.where(lane128 == 6, bc(idx1),
          jnp.where(lane128 == 7, bc(idx2), 0.0))))))))
    out_ref[...] = out

    bl = lax.broadcasted_iota(_I32, (16, 128), 1).astype(_F32) * BLK
    startc = jnp.broadcast_to(jnp.transpose(start), (16, 128))
    paddedc = jnp.broadcast_to(jnp.transpose(padded), (16, 128))
    lane16c = lax.broadcasted_iota(_I32, (16, 128), 0)
    inseg = ((bl >= startc) & (bl < startc + paddedc)
             & (lane16c < E)).astype(_F32)
    eidx = lane16c.astype(_F32) * inseg
    ones16 = jnp.ones((1, 16), _F32)
    bexp = lax.dot_general(ones16, eidx, (((1,), (0,)), ((), ())),
                           preferred_element_type=_F32)
    bact = lax.dot_general(ones16, inseg, (((1,), (0,)), ((), ())),
                           preferred_element_type=_F32)
    eye = (lax.broadcasted_iota(_I32, (16, 128), 0)
           == lax.broadcasted_iota(_I32, (16, 128), 1)).astype(_F32)
    counts128 = lax.dot_general(counts, eye, (((1,), (0,)), ((), ())),
                                preferred_element_type=_F32)
    row8 = lax.broadcasted_iota(_I32, (8, 128), 0)
    meta = jnp.where(row8 == 0, jnp.broadcast_to(bexp, (8, 128)),
           jnp.where(row8 == 1, jnp.broadcast_to(bact, (8, 128)),
           jnp.where(row8 == 2, jnp.broadcast_to(counts128, (8, 128)),
           jnp.where(row8 == 3,
                     jnp.where(lane128[0:8, :] == 0, dummy, sinkf), 0.0))))
    meta_ref[...] = meta


def _run_router(x2d, router_w):
    rw16 = jnp.zeros((16, H), _F32).at[:NSLOT].set(router_w)
    return pl.pallas_call(
        _router_body,
        out_shape=[jax.ShapeDtypeStruct((T, 128), _F32),
                   jax.ShapeDtypeStruct((8, 128), _F32),
                   jax.ShapeDtypeStruct((1, 1), _F32)],
        out_specs=[pl.BlockSpec(memory_space=pltpu.VMEM),
                   pl.BlockSpec(memory_space=pltpu.VMEM),
                   pl.BlockSpec(memory_space=pltpu.SMEM)],
    )(x2d, rw16)


def _dispatch_body(x_hbm, idx1_hbm, idx2_hbm, idx1f_hbm, idx2f_hbm,
                   w1_hbm, w2_hbm, dmy_hbm,
                   z_hbm, xd_hbm, wrow_hbm,

                   i1r, i2r, i1f, i2f, w1v, w2v, rva, rvb, dmyv, zrow, zw,
                   sema, semb, semw):
    wid = lax.axis_index("s") * NC + lax.axis_index("c")
    t0 = wid * TPW
    nch = TPW // L
    pltpu.sync_copy(idx1_hbm.at[pl.ds(wid * nch, nch)], i1r)
    pltpu.sync_copy(idx2_hbm.at[pl.ds(wid * nch, nch)], i2r)
    pltpu.sync_copy(w1_hbm.at[pl.ds(t0, TPW)], w1v)
    pltpu.sync_copy(w2_hbm.at[pl.ds(t0, TPW)], w2v)
    pltpu.sync_copy(idx1f_hbm.at[pl.ds(t0, TPW)], i1f)
    pltpu.sync_copy(idx2f_hbm.at[pl.ds(t0, TPW)], i2f)
    cpw1 = pltpu.async_copy(w1v, wrow_hbm.at[i1f], semw)
    cpw2 = pltpu.async_copy(w2v, wrow_hbm.at[i2f], semw)
    rvs = (rva, rvb)
    sems = (sema, semb)
    pltpu.sync_copy(x_hbm.at[pl.ds(t0, L)], rva)
    pend = []
    for j in range(nch):
        rv = rvs[j % 2]
        sm = sems[j % 2]
        pend.append(pltpu.async_copy(rv, xd_hbm.at[i1r.at[j]], sm))
        pend.append(pltpu.async_copy(rv, xd_hbm.at[i2r.at[j]], sm))
        if j + 1 < nch:
            if j >= 1:
                pend.pop(0).wait()
                pend.pop(0).wait()
            pltpu.sync_copy(x_hbm.at[pl.ds(t0 + (j + 1) * L, L)],
                            rvs[(j + 1) % 2])
    for cp in pend:
        cp.wait()
    cpw1.wait()
    cpw2.wait()

    @pl.when(wid == 0)
    def _dummy():
        pltpu.sync_copy(dmy_hbm, dmyv)
        pltpu.sync_copy(z_hbm, zrow)
        pltpu.sync_copy(z_hbm.at[0, pl.ds(0, L)], zw)
        pltpu.async_copy(zrow, xd_hbm.at[dmyv], semw).wait()
        pltpu.async_copy(zw, wrow_hbm.at[dmyv], semw).wait()


def _run_dispatch(x2d, idx1, idx2, w1, w2, dmy):
    mesh = plsc.VectorSubcoreMesh(core_axis_name="c", subcore_axis_name="s",
                                  num_cores=NC, num_subcores=NS)
    f = pl.kernel(
        _dispatch_body,
        out_type=[jax.ShapeDtypeStruct((PPAD, H), _F32),
                  jax.ShapeDtypeStruct((PPAD,), _F32)],
        mesh=mesh,
        compiler_params=pltpu.CompilerParams(needs_layout_passes=False),
        scratch_types=[
            pltpu.VMEM((TPW // L, L), _I32), pltpu.VMEM((TPW // L, L), _I32),
            pltpu.VMEM((TPW,), _I32), pltpu.VMEM((TPW,), _I32),
            pltpu.VMEM((TPW,), _F32), pltpu.VMEM((TPW,), _F32),
            pltpu.VMEM((L, H), _F32), pltpu.VMEM((L, H), _F32),
            pltpu.VMEM((L,), _I32),
            pltpu.VMEM((L, H), _F32),
            pltpu.VMEM((L,), _F32),
            pltpu.SemaphoreType.DMA, pltpu.SemaphoreType.DMA,
            pltpu.SemaphoreType.DMA,
        ],
    )
    return f(x2d, idx1.reshape(T // L, L), idx2.reshape(T // L, L),
             idx1, idx2, w1, w2, dmy, jnp.zeros((L, H), _F32))


def _gffn_body(bexp_ref, bact_ref, xd_ref, gw_ref, uw_ref, dw_ref, wr_ref,
               yd_ref):
    i = pl.program_id(0)

    @pl.when(bact_ref[i] == 1)
    def _():
        xb = xd_ref[...]
        g = lax.dot_general(xb, gw_ref[0], (((1,), (1,)), ((), ())),
                            preferred_element_type=_F32)
        u = lax.dot_general(xb, uw_ref[0], (((1,), (1,)), ((), ())),
                            preferred_element_type=_F32)
        h = g * lax.logistic(g) * u
        y = lax.dot_general(h, dw_ref[0], (((1,), (1,)), ((), ())),
                            preferred_element_type=_F32)
        w = jnp.transpose(wr_ref[0])
        yd_ref[...] = y * w


def _run_gffn(bexp, bact, xd, gate_w, up_w, down_w, wrow):
    wr3d = wrow[:PR].reshape(NBR, 1, BLK)
    grid_spec = pltpu.PrefetchScalarGridSpec(
        num_scalar_prefetch=2,
        grid=(NBR,),
        in_specs=[
            pl.BlockSpec((BLK, H), lambda i, be, ba: (i, 0)),
            pl.BlockSpec((1, FF, H), lambda i, be, ba: (be[i], 0, 0)),
            pl.BlockSpec((1, FF, H), lambda i, be, ba: (be[i], 0, 0)),
            pl.BlockSpec((1, H, FF), lambda i, be, ba: (be[i], 0, 0)),
            pl.BlockSpec((1, 1, BLK), lambda i, be, ba: (i, 0, 0)),
        ],
        out_specs=pl.BlockSpec((BLK, H), lambda i, be, ba: (i, 0)),
    )
    return pl.pallas_call(
        _gffn_body,
        grid_spec=grid_spec,
        out_shape=jax.ShapeDtypeStruct((PR, H), _F32),
        compiler_params=pltpu.CompilerParams(
            vmem_limit_bytes=120 * 1024 * 1024),
    )(bexp, bact, xd, gate_w, up_w, down_w, wr3d)


def _sffn_body(x_ref, gw_ref, uw_ref, dw_ref, o_ref):
    xb = x_ref[...]
    g = lax.dot_general(xb, gw_ref[...], (((1,), (1,)), ((), ())),
                        preferred_element_type=_F32)
    u = lax.dot_general(xb, uw_ref[...], (((1,), (1,)), ((), ())),
                        preferred_element_type=_F32)
    h = g * lax.logistic(g) * u
    o_ref[...] = lax.dot_general(h, dw_ref[...], (((1,), (1,)), ((), ())),
                                 preferred_element_type=_F32)


def _run_sffn(x2d, sgw, suw, sdw):
    return pl.pallas_call(
        _sffn_body,
        grid=(T // BLK,),
        in_specs=[
            pl.BlockSpec((BLK, H), lambda i: (i, 0)),
            pl.BlockSpec((FF, H), lambda i: (0, 0)),
            pl.BlockSpec((FF, H), lambda i: (0, 0)),
            pl.BlockSpec((H, FF), lambda i: (0, 0)),
        ],
        out_specs=pl.BlockSpec((BLK, H), lambda i: (i, 0)),
        out_shape=jax.ShapeDtypeStruct((T, H), _F32),
    )(x2d, sgw, suw, sdw)


def _combine_body(yds_hbm, yd_hbm, pos1_hbm, pos2_hbm, out_hbm,
                  p1a, p2a, p1b, p2b, r1a, r2a, acca, r1b, r2b, accb, sem):
    wid = lax.axis_index("s") * NC + lax.axis_index("c")
    csz = 16
    nch = TPW // csz
    bufs = ((p1a, p2a, r1a, r2a, acca), (p1b, p2b, r1b, r2b, accb))

    def fetch(ch, bi):
        p1, p2, r1, r2, acc = bufs[bi]
        t0 = wid * TPW + ch * csz
        pltpu.sync_copy(pos1_hbm.at[pl.ds(t0, csz)], p1)
        pltpu.sync_copy(pos2_hbm.at[pl.ds(t0, csz)], p2)
        cp1 = pltpu.async_copy(yd_hbm.at[p1], r1, sem)
        cp2 = pltpu.async_copy(yd_hbm.at[p2], r2, sem)
        cp3 = pltpu.async_copy(yds_hbm.at[pl.ds(t0, csz)], acc, sem)
        return (cp1, cp2, cp3)

    pend = fetch(0, 0)
    for ch in range(nch):
        bi = ch % 2
        _, _, r1, r2, acc = bufs[bi]
        for cp in pend:
            cp.wait()
        if ch + 1 < nch:
            pend = fetch(ch + 1, (ch + 1) % 2)

        def add_row(r, _):
            for k in range(H // L):
                a = (acc[r, pl.ds(k * L, L)] + r1[r, pl.ds(k * L, L)]
                     + r2[r, pl.ds(k * L, L)])
                acc[r, pl.ds(k * L, L)] = a
            return 0

        lax.fori_loop(0, csz, add_row, 0)
        t0 = wid * TPW + ch * csz
        pltpu.sync_copy(acc, out_hbm.at[pl.ds(t0, csz)])


def _run_combine(yds, yd, pos1, pos2):
    mesh = plsc.VectorSubcoreMesh(core_axis_name="c", subcore_axis_name="s",
                                  num_cores=NC, num_subcores=NS)
    f = pl.kernel(
        _combine_body,
        out_type=[jax.ShapeDtypeStruct((T, H), _F32)],
        mesh=mesh,
        compiler_params=pltpu.CompilerParams(needs_layout_passes=False),
        scratch_types=[
            pltpu.VMEM((16,), _I32), pltpu.VMEM((16,), _I32),
            pltpu.VMEM((16,), _I32), pltpu.VMEM((16,), _I32),
            pltpu.VMEM((16, H), _F32), pltpu.VMEM((16, H), _F32),
            pltpu.VMEM((16, H), _F32),
            pltpu.VMEM((16, H), _F32), pltpu.VMEM((16, H), _F32),
            pltpu.VMEM((16, H), _F32),
            pltpu.SemaphoreType.DMA,
        ],
    )
    return f(yds, yd, pos1, pos2)[0]


def kernel(hidden_states, router_w, gate_w, up_w, down_w,
           shared_gate_w, shared_up_w, shared_down_w):
    x2d = hidden_states.reshape(T, H)

    rtr, meta, z_sum = _run_router(x2d, router_w)
    w1 = rtr[:, 2]
    w2 = rtr[:, 3]
    cpos1 = rtr[:, 4].astype(_I32)
    cpos2 = rtr[:, 5].astype(_I32)
    idx1 = rtr[:, 6].astype(_I32)
    idx2 = rtr[:, 7].astype(_I32)
    bexp = meta[0, :NBR].astype(_I32)
    bact = meta[1, :NBR].astype(_I32)
    counts = meta[2, :16]
    dmy = meta[3, :16].astype(_I32)

    xd, wrow = _run_dispatch(x2d, idx1, idx2, w1, w2, dmy)
    yds = _run_sffn(x2d, shared_gate_w, shared_up_w, shared_down_w)
    yd = _run_gffn(bexp, bact, xd, gate_w, up_w, down_w, wrow)
    out2d = _run_combine(yds, yd, cpos1, cpos2)

    final_outputs = out2d.reshape(B, S, H)

    loads = jnp.concatenate([counts[:E], jnp.full((1,), float(T), _F32)])
    loads_norm = loads / (jnp.sum(loads) + 1e-6)
    ideal = 1.0 / NSLOT
    load_balance_loss = jnp.mean((loads_norm - ideal) ** 2)
    router_z_loss = z_sum[0, 0] / T
    total_aux_loss = 0.01 * load_balance_loss + 0.01 * router_z_loss
    return final_outputs, total_aux_loss
